# baseline (speedup 1.0000x reference)
import jax
import jax.numpy as jnp
from jax.experimental import pallas as pl

H = 128
BATCH_BLK = 32


def _modal_add_kernel(poi_ref, img_ref, tbl_ref, poi_out_ref, img_out_ref):
    poi_out_ref[...] = poi_ref[...] + tbl_ref[0:1, :][None]
    img_out_ref[...] = img_ref[...] + tbl_ref[1:2, :][None]


def kernel(poi_embedding, img_embedding, mod_embed_table):
    B, S_poi, h = poi_embedding.shape
    S_img = img_embedding.shape[1]
    grid = (B // BATCH_BLK,)
    return pl.pallas_call(
        _modal_add_kernel,
        grid=grid,
        in_specs=[
            pl.BlockSpec((BATCH_BLK, S_poi, h), lambda i: (i, 0, 0)),
            pl.BlockSpec((BATCH_BLK, S_img, h), lambda i: (i, 0, 0)),
            pl.BlockSpec((2, h), lambda i: (0, 0)),
        ],
        out_specs=[
            pl.BlockSpec((BATCH_BLK, S_poi, h), lambda i: (i, 0, 0)),
            pl.BlockSpec((BATCH_BLK, S_img, h), lambda i: (i, 0, 0)),
        ],
        out_shape=[
            jax.ShapeDtypeStruct(poi_embedding.shape, poi_embedding.dtype),
            jax.ShapeDtypeStruct(img_embedding.shape, img_embedding.dtype),
        ],
    )(poi_embedding, img_embedding, mod_embed_table)
